# baseline (speedup 1.0000x reference)
import functools

import jax
import jax.numpy as jnp
from jax import lax
from jax.experimental import pallas as pl
from jax.experimental.pallas import tpu as pltpu
from jax.experimental.pallas import tpu_sc as plsc

B = 16384
D = 64
DP = D // 2
NC = 2
NS = 16
L = 16
NW = NC * NS
BPW = B // NW
CH = 16
NCHK = BPW // CH

_mesh = plsc.VectorSubcoreMesh(core_axis_name="c", subcore_axis_name="s")


@functools.partial(
    pl.kernel,
    out_type=jax.ShapeDtypeStruct((B,), jnp.float32),
    mesh=_mesh,
    scratch_types=[
        pltpu.VMEM((BPW,), jnp.int32),
        pltpu.VMEM((BPW,), jnp.int32),
        pltpu.VMEM((BPW,), jnp.int32),
        pltpu.VMEM((CH, DP), jnp.int32),
        pltpu.VMEM((CH, DP), jnp.int32),
        pltpu.VMEM((CH, DP), jnp.int32),
        pltpu.VMEM((CH, DP), jnp.int32),
        pltpu.VMEM((CH, DP), jnp.int32),
        pltpu.VMEM((CH, DP), jnp.int32),
        pltpu.VMEM((BPW,), jnp.float32),
        pltpu.SemaphoreType.DMA,
        pltpu.SemaphoreType.DMA,
    ],
    compiler_params=pltpu.CompilerParams(
        needs_layout_passes=False, use_tc_tiling_on_sc=False),
)
def _bpr_kernel(uid_hbm, pid_hbm, nid_hbm, ut_hbm, it_hbm, out_hbm,
                idv_u, idv_p, idv_n, bu0, bu1, bp0, bp1, bn0, bn1,
                out_v, sem0, sem1):
    w = lax.axis_index("s") * NC + lax.axis_index("c")
    base = w * BPW

    pltpu.sync_copy(uid_hbm.at[pl.ds(base, BPW)], idv_u)
    pltpu.sync_copy(pid_hbm.at[pl.ds(base, BPW)], idv_p)
    pltpu.sync_copy(nid_hbm.at[pl.ds(base, BPW)], idv_n)

    bufs = ((bu0, bp0, bn0), (bu1, bp1, bn1))
    sems = (sem0, sem1)
    lane = lax.iota(jnp.int32, L)
    zeros = jnp.zeros((L,), jnp.int32)
    himask = jnp.full((L,), -65536, jnp.int32)
    sixteen = jnp.full((L,), 16, jnp.int32)

    def fire_chunk(c, par):
        bu, bp, bn = bufs[par]
        sem = sems[par]
        sl = pl.ds(c * CH, CH)
        qv_u = idv_u[sl]
        qv_p = idv_p[sl]
        qv_n = idv_n[sl]
        for i in range(CH):
            m = lane == i
            q_u = jnp.sum(jnp.where(m, qv_u, zeros))
            q_p = jnp.sum(jnp.where(m, qv_p, zeros))
            q_n = jnp.sum(jnp.where(m, qv_n, zeros))
            pltpu.make_async_copy(ut_hbm.at[q_u], bu.at[i], sem).start()
            pltpu.make_async_copy(it_hbm.at[q_p], bp.at[i], sem).start()
            pltpu.make_async_copy(it_hbm.at[q_n], bn.at[i], sem).start()

    def drain_chunk(par):
        bu, bp, bn = bufs[par]
        sem = sems[par]
        dummy = ut_hbm.at[pl.ds(0, CH)]
        pltpu.make_async_copy(dummy, bu, sem).wait()
        pltpu.make_async_copy(dummy, bp, sem).wait()
        pltpu.make_async_copy(dummy, bn, sem).wait()

    def unpack2(x):
        lo = plsc.bitcast(lax.shift_left(x, sixteen), jnp.float32)
        hi = plsc.bitcast(lax.bitwise_and(x, himask), jnp.float32)
        return lo, hi

    def compute_chunk(c, par):
        bu, bp, bn = bufs[par]

        def dbody(d, acc):
            dv = jnp.full((L,), d, jnp.int32)
            xu = plsc.load_gather(bu, [lane, dv])
            xp = plsc.load_gather(bp, [lane, dv])
            xn = plsc.load_gather(bn, [lane, dv])
            ulo, uhi = unpack2(xu)
            plo, phi = unpack2(xp)
            nlo, nhi = unpack2(xn)
            return acc + ulo * (plo - nlo) + uhi * (phi - nhi)

        acc = lax.fori_loop(0, DP, dbody, jnp.zeros((L,), jnp.float32),
                            unroll=8)
        out_v[pl.ds(c * CH, CH)] = acc

    fire_chunk(0, 0)
    fire_chunk(1, 1)

    def outer(t, carry):
        c0 = t * 2

        drain_chunk(0)
        compute_chunk(c0, 0)

        @pl.when(t < NCHK // 2 - 1)
        def _():
            fire_chunk(c0 + 2, 0)

        drain_chunk(1)
        compute_chunk(c0 + 1, 1)

        @pl.when(t < NCHK // 2 - 1)
        def _():
            fire_chunk(c0 + 3, 1)

        return carry

    lax.fori_loop(0, NCHK // 2, outer, 0)

    pltpu.sync_copy(out_v, out_hbm.at[pl.ds(base, BPW)])


def _pack_bf16(table):
    n = table.shape[0]
    tb = table.astype(jnp.bfloat16).reshape(n, DP, 2)
    return lax.bitcast_convert_type(tb, jnp.int32)


def kernel(user_ids, pos_item_ids, neg_item_ids, user_table, item_table):
    uid = user_ids.astype(jnp.int32)
    pid = pos_item_ids.astype(jnp.int32)
    nid = neg_item_ids.astype(jnp.int32)
    return _bpr_kernel(uid, pid, nid,
                       _pack_bf16(user_table), _pack_bf16(item_table))


# speedup vs baseline: 3.2768x; 3.2768x over previous
import functools

import jax
import jax.numpy as jnp
from jax import lax
from jax.experimental import pallas as pl
from jax.experimental.pallas import tpu as pltpu
from jax.experimental.pallas import tpu_sc as plsc

B = 16384
D = 64
NC = 2
NS = 16
L = 16
NW = NC * NS
BPW = B // NW
CH = 16
NCHK = BPW // CH

_mesh = plsc.VectorSubcoreMesh(core_axis_name="c", subcore_axis_name="s")


def _extract_scalars(idv, sl, lane, zeros):
    qv = idv[sl]
    out = []
    for i in range(L):
        out.append(jnp.sum(jnp.where(lane == i, qv, zeros)))
    return out


@functools.partial(
    pl.kernel,
    out_type=jax.ShapeDtypeStruct((B, D), jnp.float32),
    mesh=_mesh,
    scratch_types=[
        pltpu.VMEM((BPW,), jnp.int32),
        pltpu.VMEM((BPW,), jnp.int32),
        pltpu.VMEM((CH, D), jnp.float32),
        pltpu.VMEM((CH, D), jnp.float32),
        pltpu.VMEM((CH, D), jnp.float32),
        pltpu.VMEM((CH, D), jnp.float32),
        pltpu.VMEM((CH, D), jnp.float32),
        pltpu.SemaphoreType.DMA,
        pltpu.SemaphoreType.DMA,
    ],
    compiler_params=pltpu.CompilerParams(
        needs_layout_passes=False, use_tc_tiling_on_sc=False),
)
def _diff_kernel(pid_hbm, nid_hbm, it_hbm, out_hbm,
                 idv_p, idv_n, bp0, bp1, bn0, bn1, dbuf, sem0, sem1):
    w = lax.axis_index("s") * NC + lax.axis_index("c")
    base = w * BPW

    pltpu.sync_copy(pid_hbm.at[pl.ds(base, BPW)], idv_p)
    pltpu.sync_copy(nid_hbm.at[pl.ds(base, BPW)], idv_n)

    bufs = ((bp0, bn0), (bp1, bn1))
    sems = (sem0, sem1)
    lane = lax.iota(jnp.int32, L)
    zeros = jnp.zeros((L,), jnp.int32)

    def fire_chunk(c, par):
        bp, bn = bufs[par]
        sem = sems[par]
        sl = pl.ds(c * CH, CH)
        q_p = _extract_scalars(idv_p, sl, lane, zeros)
        q_n = _extract_scalars(idv_n, sl, lane, zeros)
        for i in range(CH):
            pltpu.make_async_copy(it_hbm.at[q_p[i]], bp.at[i], sem).start()
            pltpu.make_async_copy(it_hbm.at[q_n[i]], bn.at[i], sem).start()

    def drain_chunk(par):
        bp, bn = bufs[par]
        sem = sems[par]
        dummy = it_hbm.at[pl.ds(0, CH)]
        pltpu.make_async_copy(dummy, bp, sem).wait()
        pltpu.make_async_copy(dummy, bn, sem).wait()

    def compute_chunk(c, par):
        bp, bn = bufs[par]
        for i in range(CH):
            for k in range(D // L):
                ksl = pl.ds(k * L, L)
                dbuf[i, ksl] = bp[i, ksl] - bn[i, ksl]
        pltpu.sync_copy(dbuf, out_hbm.at[pl.ds(base + c * CH, CH)])

    fire_chunk(0, 0)
    fire_chunk(1, 1)

    def outer(t, carry):
        c0 = t * 2

        drain_chunk(0)
        compute_chunk(c0, 0)

        @pl.when(t < NCHK // 2 - 1)
        def _():
            fire_chunk(c0 + 2, 0)

        drain_chunk(1)
        compute_chunk(c0 + 1, 1)

        @pl.when(t < NCHK // 2 - 1)
        def _():
            fire_chunk(c0 + 3, 1)

        return carry

    lax.fori_loop(0, NCHK // 2, outer, 0)


@functools.partial(
    pl.kernel,
    out_type=jax.ShapeDtypeStruct((B,), jnp.float32),
    mesh=_mesh,
    scratch_types=[
        pltpu.VMEM((BPW,), jnp.int32),
        pltpu.VMEM((CH, D), jnp.float32),
        pltpu.VMEM((CH, D), jnp.float32),
        pltpu.VMEM((CH, D), jnp.float32),
        pltpu.VMEM((CH, D), jnp.float32),
        pltpu.VMEM((BPW,), jnp.float32),
        pltpu.SemaphoreType.DMA,
        pltpu.SemaphoreType.DMA,
    ],
    compiler_params=pltpu.CompilerParams(needs_layout_passes=False),
)
def _dot_kernel(uid_hbm, ut_hbm, h_hbm, out_hbm,
                idv_u, bu0, bu1, bh0, bh1, out_v, sem0, sem1):
    w = lax.axis_index("s") * NC + lax.axis_index("c")
    base = w * BPW

    pltpu.sync_copy(uid_hbm.at[pl.ds(base, BPW)], idv_u)

    bufs = ((bu0, bh0), (bu1, bh1))
    sems = (sem0, sem1)
    lane = lax.iota(jnp.int32, L)
    zeros = jnp.zeros((L,), jnp.int32)

    def fire_chunk(c, par):
        bu, bh = bufs[par]
        sem = sems[par]
        sl = pl.ds(c * CH, CH)
        q_u = _extract_scalars(idv_u, sl, lane, zeros)
        for i in range(CH):
            pltpu.make_async_copy(ut_hbm.at[q_u[i]], bu.at[i], sem).start()
        pltpu.make_async_copy(h_hbm.at[pl.ds(base + c * CH, CH)], bh,
                              sem).start()

    def drain_chunk(par):
        bu, bh = bufs[par]
        sem = sems[par]
        dummy = ut_hbm.at[pl.ds(0, CH)]
        pltpu.make_async_copy(dummy, bu, sem).wait()
        pltpu.make_async_copy(dummy, bh, sem).wait()

    def compute_chunk(c, par):
        bu, bh = bufs[par]

        def dbody(d, acc):
            dv = jnp.full((L,), d, jnp.int32)
            u = plsc.load_gather(bu, [lane, dv])
            h = plsc.load_gather(bh, [lane, dv])
            return acc + u * h

        acc = lax.fori_loop(0, D, dbody, jnp.zeros((L,), jnp.float32),
                            unroll=8)
        out_v[pl.ds(c * CH, CH)] = acc

    fire_chunk(0, 0)
    fire_chunk(1, 1)

    def outer(t, carry):
        c0 = t * 2

        drain_chunk(0)
        compute_chunk(c0, 0)

        @pl.when(t < NCHK // 2 - 1)
        def _():
            fire_chunk(c0 + 2, 0)

        drain_chunk(1)
        compute_chunk(c0 + 1, 1)

        @pl.when(t < NCHK // 2 - 1)
        def _():
            fire_chunk(c0 + 3, 1)

        return carry

    lax.fori_loop(0, NCHK // 2, outer, 0)

    pltpu.sync_copy(out_v, out_hbm.at[pl.ds(base, BPW)])


def kernel(user_ids, pos_item_ids, neg_item_ids, user_table, item_table):
    uid = user_ids.astype(jnp.int32)
    pid = pos_item_ids.astype(jnp.int32)
    nid = neg_item_ids.astype(jnp.int32)
    diff = _diff_kernel(pid, nid, item_table)
    return _dot_kernel(uid, user_table, diff)


# speedup vs baseline: 4.3921x; 1.3403x over previous
import functools

import jax
import jax.numpy as jnp
from jax import lax
from jax.experimental import pallas as pl
from jax.experimental.pallas import tpu as pltpu
from jax.experimental.pallas import tpu_sc as plsc

B = 16384
D = 64
NC = 2
NS = 16
L = 16
NW = NC * NS
BPW = B // NW
CH = 16
NCHK = BPW // CH

_mesh = plsc.VectorSubcoreMesh(core_axis_name="c", subcore_axis_name="s")


@functools.partial(
    pl.kernel,
    out_type=jax.ShapeDtypeStruct((B,), jnp.float32),
    mesh=_mesh,
    scratch_types=[
        pltpu.VMEM((BPW,), jnp.int32),
        pltpu.VMEM((BPW,), jnp.int32),
        pltpu.VMEM((BPW,), jnp.int32),
        pltpu.VMEM((CH, D), jnp.float32),
        pltpu.VMEM((CH, D), jnp.float32),
        pltpu.VMEM((CH, D), jnp.float32),
        pltpu.VMEM((CH, D), jnp.float32),
        pltpu.VMEM((CH, D), jnp.float32),
        pltpu.VMEM((CH, D), jnp.float32),
        pltpu.VMEM((BPW,), jnp.float32),
        pltpu.SemaphoreType.DMA,
        pltpu.SemaphoreType.DMA,
    ],
    compiler_params=pltpu.CompilerParams(needs_layout_passes=False),
)
def _bpr_kernel(uid_hbm, pid_hbm, nid_hbm, ut_hbm, it_hbm, out_hbm,
                idv_u, idv_p, idv_n, bu0, bu1, bp0, bp1, bn0, bn1,
                out_v, sem0, sem1):
    w = lax.axis_index("s") * NC + lax.axis_index("c")
    base = w * BPW

    pltpu.sync_copy(uid_hbm.at[pl.ds(base, BPW)], idv_u)
    pltpu.sync_copy(pid_hbm.at[pl.ds(base, BPW)], idv_p)
    pltpu.sync_copy(nid_hbm.at[pl.ds(base, BPW)], idv_n)

    bufs = ((bu0, bp0, bn0), (bu1, bp1, bn1))
    sems = (sem0, sem1)
    lane = lax.iota(jnp.int32, L)
    zeros = jnp.zeros((L,), jnp.int32)

    def fire_chunk(c, par):
        bu, bp, bn = bufs[par]
        sem = sems[par]
        sl = pl.ds(c * CH, CH)
        qv_u = idv_u[sl]
        qv_p = idv_p[sl]
        qv_n = idv_n[sl]
        for i in range(CH):
            m = lane == i
            q_u = jnp.sum(jnp.where(m, qv_u, zeros))
            q_p = jnp.sum(jnp.where(m, qv_p, zeros))
            q_n = jnp.sum(jnp.where(m, qv_n, zeros))
            pltpu.make_async_copy(ut_hbm.at[q_u], bu.at[i], sem).start()
            pltpu.make_async_copy(it_hbm.at[q_p], bp.at[i], sem).start()
            pltpu.make_async_copy(it_hbm.at[q_n], bn.at[i], sem).start()

    def drain_chunk(par):
        bu, bp, bn = bufs[par]
        sem = sems[par]
        dummy = ut_hbm.at[pl.ds(0, CH)]
        pltpu.make_async_copy(dummy, bu, sem).wait()
        pltpu.make_async_copy(dummy, bp, sem).wait()
        pltpu.make_async_copy(dummy, bn, sem).wait()

    def compute_chunk(c, par):
        bu, bp, bn = bufs[par]

        def dbody(d, acc):
            dv = jnp.full((L,), d, jnp.int32)
            u = plsc.load_gather(bu, [lane, dv])
            p = plsc.load_gather(bp, [lane, dv])
            n = plsc.load_gather(bn, [lane, dv])
            return acc + u * (p - n)

        acc = lax.fori_loop(0, D, dbody, jnp.zeros((L,), jnp.float32),
                            unroll=8)
        out_v[pl.ds(c * CH, CH)] = acc

    fire_chunk(0, 0)
    fire_chunk(1, 1)

    def outer(t, carry):
        c0 = t * 2

        drain_chunk(0)
        compute_chunk(c0, 0)

        @pl.when(t < NCHK // 2 - 1)
        def _():
            fire_chunk(c0 + 2, 0)

        drain_chunk(1)
        compute_chunk(c0 + 1, 1)

        @pl.when(t < NCHK // 2 - 1)
        def _():
            fire_chunk(c0 + 3, 1)

        return carry

    lax.fori_loop(0, NCHK // 2, outer, 0)

    pltpu.sync_copy(out_v, out_hbm.at[pl.ds(base, BPW)])


def kernel(user_ids, pos_item_ids, neg_item_ids, user_table, item_table):
    uid = user_ids.astype(jnp.int32)
    pid = pos_item_ids.astype(jnp.int32)
    nid = neg_item_ids.astype(jnp.int32)
    return _bpr_kernel(uid, pid, nid, user_table, item_table)


# speedup vs baseline: 5.7664x; 1.3129x over previous
import functools

import jax
import jax.numpy as jnp
from jax import lax
from jax.experimental import pallas as pl
from jax.experimental.pallas import tpu as pltpu
from jax.experimental.pallas import tpu_sc as plsc

B = 16384
D = 64
NC = 2
NS = 16
L = 16
NW = NC * NS
BPW = B // NW
CH = 16
NCHK = BPW // CH

_mesh = plsc.VectorSubcoreMesh(core_axis_name="c", subcore_axis_name="s")


@functools.partial(
    pl.kernel,
    out_type=jax.ShapeDtypeStruct((B,), jnp.float32),
    mesh=_mesh,
    scratch_types=[
        pltpu.VMEM((BPW,), jnp.int32),
        pltpu.VMEM((BPW,), jnp.int32),
        pltpu.VMEM((BPW,), jnp.int32),
        pltpu.VMEM((CH, D), jnp.float32),
        pltpu.VMEM((CH, D), jnp.float32),
        pltpu.VMEM((CH, D), jnp.float32),
        pltpu.VMEM((CH, D), jnp.float32),
        pltpu.VMEM((CH, D), jnp.float32),
        pltpu.VMEM((CH, D), jnp.float32),
        pltpu.VMEM((BPW,), jnp.float32),
        pltpu.SemaphoreType.DMA,
        pltpu.SemaphoreType.DMA,
    ],
    compiler_params=pltpu.CompilerParams(needs_layout_passes=False),
)
def _bpr_kernel(uid_hbm, pid_hbm, nid_hbm, ut_hbm, it_hbm, out_hbm,
                idv_u, idv_p, idv_n, bu0, bu1, bp0, bp1, bn0, bn1,
                out_v, sem0, sem1):
    w = lax.axis_index("s") * NC + lax.axis_index("c")
    base = w * BPW

    pltpu.sync_copy(uid_hbm.at[pl.ds(base, BPW)], idv_u)
    pltpu.sync_copy(pid_hbm.at[pl.ds(base, BPW)], idv_p)
    pltpu.sync_copy(nid_hbm.at[pl.ds(base, BPW)], idv_n)

    bufs = ((bu0, bp0, bn0), (bu1, bp1, bn1))
    sems = (sem0, sem1)
    lane = lax.iota(jnp.int32, L)
    zeros = jnp.zeros((L,), jnp.int32)

    def fire_chunk(c, par):
        bu, bp, bn = bufs[par]
        sem = sems[par]
        sl = pl.ds(c * CH, CH)
        qv_u = idv_u[sl]
        qv_p = idv_p[sl]
        qv_n = idv_n[sl]
        for i in range(CH):
            q_u = qv_u[i]
            q_p = qv_p[i]
            q_n = qv_n[i]
            pltpu.make_async_copy(ut_hbm.at[q_u], bu.at[i], sem).start()
            pltpu.make_async_copy(it_hbm.at[q_p], bp.at[i], sem).start()
            pltpu.make_async_copy(it_hbm.at[q_n], bn.at[i], sem).start()

    def drain_chunk(par):
        bu, bp, bn = bufs[par]
        sem = sems[par]
        dummy = ut_hbm.at[pl.ds(0, CH)]
        pltpu.make_async_copy(dummy, bu, sem).wait()
        pltpu.make_async_copy(dummy, bp, sem).wait()
        pltpu.make_async_copy(dummy, bn, sem).wait()

    def compute_chunk(c, par):
        bu, bp, bn = bufs[par]

        def dbody(d, acc):
            dv = jnp.full((L,), d, jnp.int32)
            u = plsc.load_gather(bu, [lane, dv])
            p = plsc.load_gather(bp, [lane, dv])
            n = plsc.load_gather(bn, [lane, dv])
            return acc + u * (p - n)

        acc = lax.fori_loop(0, D, dbody, jnp.zeros((L,), jnp.float32),
                            unroll=8)
        out_v[pl.ds(c * CH, CH)] = acc

    fire_chunk(0, 0)
    fire_chunk(1, 1)

    def outer(t, carry):
        c0 = t * 2

        drain_chunk(0)
        compute_chunk(c0, 0)

        @pl.when(t < NCHK // 2 - 1)
        def _():
            fire_chunk(c0 + 2, 0)

        drain_chunk(1)
        compute_chunk(c0 + 1, 1)

        @pl.when(t < NCHK // 2 - 1)
        def _():
            fire_chunk(c0 + 3, 1)

        return carry

    lax.fori_loop(0, NCHK // 2, outer, 0)

    pltpu.sync_copy(out_v, out_hbm.at[pl.ds(base, BPW)])


def kernel(user_ids, pos_item_ids, neg_item_ids, user_table, item_table):
    uid = user_ids.astype(jnp.int32)
    pid = pos_item_ids.astype(jnp.int32)
    nid = neg_item_ids.astype(jnp.int32)
    return _bpr_kernel(uid, pid, nid, user_table, item_table)
